# Initial kernel scaffold; baseline (speedup 1.0000x reference)
import functools

import jax
import jax.numpy as jnp
from jax import lax
from jax.experimental import pallas as pl
from jax.experimental.pallas import tpu as pltpu
from jax.experimental.pallas import tpu_sc as plsc

_NC = 2
_NS = 16
_NW = _NC * _NS
_CW = 80


def _sc_aggregate(n, e, d, x, col2, row2, ev2, zeros):
    n_chunks = e // _CW
    cpw = n_chunks // _NW
    rpt = n // _NS

    mesh = plsc.VectorSubcoreMesh(core_axis_name="c", subcore_axis_name="s")

    @functools.partial(
        pl.kernel,
        out_type=jax.ShapeDtypeStruct((_NC, n, d), jnp.float32),
        mesh=mesh,
        scratch_types=[
            pltpu.VMEM((cpw, _CW), jnp.int32),
            pltpu.VMEM((cpw, _CW), jnp.int32),
            pltpu.VMEM((cpw, _CW), jnp.float32),
            pltpu.VMEM((_CW, d), jnp.float32),
            pltpu.VMEM_SHARED((n, d), jnp.float32),
            pltpu.SemaphoreType.DMA,
        ],
    )
    def body(x_hbm, col_hbm, row_hbm, ev_hbm, z_hbm, out_hbm,
             col_v, row_v, ev_v, rows_v, agg_sh, sem):
        c = lax.axis_index("c")
        s = lax.axis_index("s")
        wid = s * _NC + c
        base = wid * cpw

        pltpu.sync_copy(z_hbm.at[pl.ds(s * rpt, rpt)],
                        agg_sh.at[pl.ds(s * rpt, rpt)])
        pltpu.sync_copy(col_hbm.at[pl.ds(base, cpw)], col_v)
        pltpu.sync_copy(row_hbm.at[pl.ds(base, cpw)], row_v)
        pltpu.sync_copy(ev_hbm.at[pl.ds(base, cpw)], ev_v)
        plsc.subcore_barrier()

        def chunk_body(j, carry):
            pltpu.async_copy(x_hbm.at[col_v.at[j]], rows_v, sem).wait()

            def scale_body(ei, carry2):
                jv = jnp.full((16,), j, dtype=jnp.int32)
                ev16 = plsc.load_gather(
                    ev_v, [jv, jnp.full((16,), ei, dtype=jnp.int32)])
                for r in range(d // 16):
                    sl = pl.ds(r * 16, 16)
                    rows_v[ei, sl] = rows_v[ei, sl] * ev16
                return carry2

            lax.fori_loop(0, _CW, scale_body, 0)

            pltpu.sync_copy(rows_v, agg_sh.at[row_v.at[j]], add=True)
            return carry

        lax.fori_loop(0, cpw, chunk_body, 0)
        plsc.subcore_barrier()

        pltpu.sync_copy(agg_sh.at[pl.ds(s * rpt, rpt)],
                        out_hbm.at[c, pl.ds(s * rpt, rpt)])

    return body(x, col2, row2, ev2, zeros)


def _linear_body(p_ref, w_ref, b_ref, o_ref):
    s = p_ref[0] + p_ref[1]
    acc = lax.dot_general(s, w_ref[...], (((1,), (1,)), ((), ())),
                          preferred_element_type=jnp.float32)
    o_ref[...] = acc + b_ref[...]


def _tc_linear(n, d, partials, w, b):
    rb = 1000
    return pl.pallas_call(
        _linear_body,
        grid=(n // rb,),
        in_specs=[
            pl.BlockSpec((_NC, rb, d), lambda i: (0, i, 0)),
            pl.BlockSpec((d, d), lambda i: (0, 0)),
            pl.BlockSpec((1, d), lambda i: (0, 0)),
        ],
        out_specs=pl.BlockSpec((rb, d), lambda i: (i, 0)),
        out_shape=jax.ShapeDtypeStruct((n, d), jnp.float32),
    )(partials, w, b.reshape(1, d))


def kernel(x, edge_index, edge_values, W, b):
    n, d = x.shape
    e = edge_values.shape[0]
    row2 = edge_index[0].reshape(e // _CW, _CW)
    col2 = edge_index[1].reshape(e // _CW, _CW)
    ev2 = edge_values.reshape(e // _CW, _CW)
    zeros = jnp.zeros((n, d), jnp.float32)
    partials = _sc_aggregate(n, e, d, x, col2, row2, ev2, zeros)
    return _tc_linear(n, d, partials, W, b)


# speedup vs baseline: 3.7257x; 3.7257x over previous
import functools

import jax
import jax.numpy as jnp
from jax import lax
from jax.experimental import pallas as pl
from jax.experimental.pallas import tpu as pltpu
from jax.experimental.pallas import tpu_sc as plsc


def _bcast16(vec, lane):
    idx = jnp.full((16, 1), lane, dtype=jnp.int32)
    dnums = lax.GatherDimensionNumbers(
        offset_dims=(), collapsed_slice_dims=(0,), start_index_map=(0,))
    return lax.gather(vec, idx, dnums, (1,),
                      mode=lax.GatherScatterMode.PROMISE_IN_BOUNDS)


_NC = 2
_NS = 16
_CW = 80


def _sc_aggregate(npad, d, x2, col3, row3, ev3, zeros):
    cps = col3.shape[1]
    rpt = npad // _NS
    dh = d // _NC

    mesh = plsc.VectorSubcoreMesh(core_axis_name="c", subcore_axis_name="s")

    @functools.partial(
        pl.kernel,
        out_type=jax.ShapeDtypeStruct((_NC, npad, dh), jnp.float32),
        mesh=mesh,
        scratch_types=[
            pltpu.VMEM((cps, _CW), jnp.int32),
            pltpu.VMEM((cps, _CW), jnp.int32),
            pltpu.VMEM((cps, _CW), jnp.float32),
            pltpu.VMEM((_CW, dh), jnp.float32),
            pltpu.VMEM_SHARED((npad, dh), jnp.float32),
            pltpu.SemaphoreType.DMA,
        ],
        compiler_params=pltpu.CompilerParams(use_tc_tiling_on_sc=False),
    )
    def body(x_hbm, col_hbm, row_hbm, ev_hbm, z_hbm, out_hbm,
             col_v, row_v, ev_v, rows_v, agg_sh, sem):
        c = lax.axis_index("c")
        s = lax.axis_index("s")

        pltpu.sync_copy(z_hbm.at[pl.ds(s * rpt, rpt)],
                        agg_sh.at[pl.ds(s * rpt, rpt)])
        pltpu.sync_copy(col_hbm.at[s], col_v)
        pltpu.sync_copy(row_hbm.at[s], row_v)
        pltpu.sync_copy(ev_hbm.at[s], ev_v)
        plsc.subcore_barrier()

        def chunk_body(j, carry):
            pltpu.async_copy(x_hbm.at[c].at[col_v.at[j]], rows_v, sem).wait()

            def scale_body(ei, carry2):
                ev16 = ev_v[j, pl.ds((ei // 16) * 16, 16)]
                evb = _bcast16(ev16, ei % 16)
                for r in range(dh // 16):
                    sl = pl.ds(r * 16, 16)
                    rows_v[ei, sl] = rows_v[ei, sl] * evb
                return carry2

            lax.fori_loop(0, _CW, scale_body, 0)

            pltpu.sync_copy(rows_v, agg_sh.at[row_v.at[j]], add=True)
            return carry

        lax.fori_loop(0, cps, chunk_body, 0)
        plsc.subcore_barrier()

        pltpu.sync_copy(agg_sh.at[pl.ds(s * rpt, rpt)],
                        out_hbm.at[c, pl.ds(s * rpt, rpt)])

    return body(x2, col3, row3, ev3, zeros)


def _linear_body(p_ref, w_ref, b_ref, o_ref):
    s = jnp.concatenate([p_ref[0], p_ref[1]], axis=1)
    acc = lax.dot_general(s, w_ref[...], (((1,), (1,)), ((), ())),
                          preferred_element_type=jnp.float32)
    o_ref[...] = acc + b_ref[...]


def _tc_linear(n, d, partials, w, b):
    rb = 1000
    dh = d // _NC
    return pl.pallas_call(
        _linear_body,
        grid=(n // rb,),
        in_specs=[
            pl.BlockSpec((_NC, rb, dh), lambda i: (0, i, 0)),
            pl.BlockSpec((d, d), lambda i: (0, 0)),
            pl.BlockSpec((1, d), lambda i: (0, 0)),
        ],
        out_specs=pl.BlockSpec((rb, d), lambda i: (i, 0)),
        out_shape=jax.ShapeDtypeStruct((n, d), jnp.float32),
    )(partials, w, b.reshape(1, d))


def kernel(x, edge_index, edge_values, W, b):
    n, d = x.shape
    e = edge_values.shape[0]
    dh = d // _NC
    cps = e // (_NS * _CW)
    npad = ((n + _NS * 8 - 1) // (_NS * 8)) * (_NS * 8)
    row3 = edge_index[0].reshape(_NS, cps, _CW)
    col3 = edge_index[1].reshape(_NS, cps, _CW)
    ev3 = edge_values.reshape(_NS, cps, _CW)
    x2 = jnp.stack([x[:, :dh], x[:, dh:]])
    zeros = jnp.zeros((npad, dh), jnp.float32)
    partials = _sc_aggregate(npad, d, x2, col3, row3, ev3, zeros)
    return _tc_linear(n, d, partials, W, b)


# speedup vs baseline: 4.7064x; 1.2632x over previous
import functools

import jax
import jax.numpy as jnp
from jax import lax
from jax.experimental import pallas as pl
from jax.experimental.pallas import tpu as pltpu
from jax.experimental.pallas import tpu_sc as plsc


def _bcast16(vec, lane):
    idx = jnp.full((16, 1), lane, dtype=jnp.int32)
    dnums = lax.GatherDimensionNumbers(
        offset_dims=(), collapsed_slice_dims=(0,), start_index_map=(0,))
    return lax.gather(vec, idx, dnums, (1,),
                      mode=lax.GatherScatterMode.PROMISE_IN_BOUNDS)


_NC = 2
_NS = 16
_CW = 80


def _sc_aggregate(npad, d, x2, col3, row3, ev3, zeros):
    cps = col3.shape[1]
    rpt = npad // _NS
    dh = d // _NC

    mesh = plsc.VectorSubcoreMesh(core_axis_name="c", subcore_axis_name="s")

    @functools.partial(
        pl.kernel,
        out_type=jax.ShapeDtypeStruct((_NC, npad, dh), jnp.float32),
        mesh=mesh,
        scratch_types=[
            pltpu.VMEM((cps, _CW), jnp.int32),
            pltpu.VMEM((cps, _CW), jnp.int32),
            pltpu.VMEM((cps, _CW), jnp.float32),
            pltpu.VMEM((_CW, dh), jnp.float32),
            pltpu.VMEM_SHARED((npad, dh), jnp.float32),
            pltpu.SemaphoreType.DMA,
        ],
        compiler_params=pltpu.CompilerParams(use_tc_tiling_on_sc=False),
    )
    def body(x_hbm, col_hbm, row_hbm, ev_hbm, z_hbm, out_hbm,
             col_v, row_v, ev_v, rows_v, agg_sh, sem):
        c = lax.axis_index("c")
        s = lax.axis_index("s")

        pltpu.sync_copy(z_hbm.at[pl.ds(s * rpt, rpt)],
                        agg_sh.at[pl.ds(s * rpt, rpt)])
        pltpu.sync_copy(col_hbm.at[s], col_v)
        pltpu.sync_copy(row_hbm.at[s], row_v)
        pltpu.sync_copy(ev_hbm.at[s], ev_v)
        plsc.subcore_barrier()

        def chunk_body(j, carry):
            pltpu.async_copy(x_hbm.at[c].at[col_v.at[j]], rows_v, sem).wait()

            for blk in range(_CW // 16):
                ev16 = ev_v[j, pl.ds(blk * 16, 16)]
                for lane in range(16):
                    evb = _bcast16(ev16, lane)
                    ei = blk * 16 + lane
                    for r in range(dh // 16):
                        sl = pl.ds(r * 16, 16)
                        rows_v[ei, sl] = rows_v[ei, sl] * evb

            pltpu.sync_copy(rows_v, agg_sh.at[row_v.at[j]], add=True)
            return carry

        lax.fori_loop(0, cps, chunk_body, 0)
        plsc.subcore_barrier()

        pltpu.sync_copy(agg_sh.at[pl.ds(s * rpt, rpt)],
                        out_hbm.at[c, pl.ds(s * rpt, rpt)])

    return body(x2, col3, row3, ev3, zeros)


def _linear_body(p_ref, w_ref, b_ref, o_ref):
    s = jnp.concatenate([p_ref[0], p_ref[1]], axis=1)
    acc = lax.dot_general(s, w_ref[...], (((1,), (1,)), ((), ())),
                          preferred_element_type=jnp.float32)
    o_ref[...] = acc + b_ref[...]


def _tc_linear(n, d, partials, w, b):
    rb = 1000
    dh = d // _NC
    return pl.pallas_call(
        _linear_body,
        grid=(n // rb,),
        in_specs=[
            pl.BlockSpec((_NC, rb, dh), lambda i: (0, i, 0)),
            pl.BlockSpec((d, d), lambda i: (0, 0)),
            pl.BlockSpec((1, d), lambda i: (0, 0)),
        ],
        out_specs=pl.BlockSpec((rb, d), lambda i: (i, 0)),
        out_shape=jax.ShapeDtypeStruct((n, d), jnp.float32),
    )(partials, w, b.reshape(1, d))


def kernel(x, edge_index, edge_values, W, b):
    n, d = x.shape
    e = edge_values.shape[0]
    dh = d // _NC
    cps = e // (_NS * _CW)
    npad = ((n + _NS * 8 - 1) // (_NS * 8)) * (_NS * 8)
    row3 = edge_index[0].reshape(_NS, cps, _CW)
    col3 = edge_index[1].reshape(_NS, cps, _CW)
    ev3 = edge_values.reshape(_NS, cps, _CW)
    x2 = jnp.stack([x[:, :dh], x[:, dh:]])
    zeros = jnp.zeros((npad, dh), jnp.float32)
    partials = _sc_aggregate(npad, d, x2, col3, row3, ev3, zeros)
    return _tc_linear(n, d, partials, W, b)


# speedup vs baseline: 8.5339x; 1.8132x over previous
import functools

import jax
import jax.numpy as jnp
from jax import lax
from jax.experimental import pallas as pl
from jax.experimental.pallas import tpu as pltpu
from jax.experimental.pallas import tpu_sc as plsc


def _bcast16(vec, lane):
    idx = jnp.full((16, 1), lane, dtype=jnp.int32)
    dnums = lax.GatherDimensionNumbers(
        offset_dims=(), collapsed_slice_dims=(0,), start_index_map=(0,))
    return lax.gather(vec, idx, dnums, (1,),
                      mode=lax.GatherScatterMode.PROMISE_IN_BOUNDS)


_NC = 2
_NS = 16
_CW = 80


def _sc_aggregate(npad, d, x2, col3, row3, ev3, zeros):
    cps = col3.shape[1]
    rpt = npad // _NS
    dh = d // _NC

    mesh = plsc.VectorSubcoreMesh(core_axis_name="c", subcore_axis_name="s")
    nbuf = 5
    assert cps % nbuf == 0

    @functools.partial(
        pl.kernel,
        out_type=jax.ShapeDtypeStruct((_NC, npad, dh), jnp.float32),
        mesh=mesh,
        scratch_types=[
            pltpu.VMEM((cps, _CW), jnp.int32),
            pltpu.VMEM((cps, _CW), jnp.int32),
            pltpu.VMEM((cps, _CW), jnp.float32),
            pltpu.VMEM((nbuf, _CW, dh), jnp.float32),
            pltpu.VMEM_SHARED((npad, dh), jnp.float32),
            pltpu.SemaphoreType.DMA((nbuf,)),
            pltpu.SemaphoreType.DMA((nbuf,)),
        ],
        compiler_params=pltpu.CompilerParams(use_tc_tiling_on_sc=False),
    )
    def body(x_hbm, col_hbm, row_hbm, ev_hbm, z_hbm, out_hbm,
             col_v, row_v, ev_v, rows_v, agg_sh, gsems, ssems):
        c = lax.axis_index("c")
        s = lax.axis_index("s")

        pltpu.sync_copy(z_hbm.at[pl.ds(s * rpt, rpt)],
                        agg_sh.at[pl.ds(s * rpt, rpt)])
        pltpu.sync_copy(col_hbm.at[s], col_v)
        pltpu.sync_copy(row_hbm.at[s], row_v)
        pltpu.sync_copy(ev_hbm.at[s], ev_v)
        plsc.subcore_barrier()

        def g_start(j, b):
            pltpu.async_copy(x_hbm.at[c].at[col_v.at[j]], rows_v.at[b],
                             gsems.at[b])

        def g_wait(b):
            pltpu.make_async_copy(x_hbm.at[c].at[pl.ds(0, _CW)],
                                  rows_v.at[b], gsems.at[b]).wait()

        def s_start(j, b):
            pltpu.async_copy(rows_v.at[b], agg_sh.at[row_v.at[j]],
                             ssems.at[b], add=True)

        def s_wait(b):
            pltpu.make_async_copy(x_hbm.at[c].at[pl.ds(0, _CW)],
                                  rows_v.at[b], ssems.at[b]).wait()

        for b in range(nbuf):
            g_start(b, b)

        def group_body(p, carry):
            for b in range(nbuf):
                j = p * nbuf + b
                g_wait(b)
                rbuf = rows_v.at[b]

                for blk in range(_CW // 16):
                    ev16 = ev_v[j, pl.ds(blk * 16, 16)]
                    for lane in range(16):
                        evb = _bcast16(ev16, lane)
                        ei = blk * 16 + lane
                        for r in range(dh // 16):
                            sl = pl.ds(r * 16, 16)
                            rbuf[ei, sl] = rbuf[ei, sl] * evb

                s_start(j, b)

                jb = j - 1
                pb = (b - 1) % nbuf

                @pl.when(jb >= 0)
                def _():
                    s_wait(pb)

                    @pl.when(jb + nbuf < cps)
                    def _():
                        g_start(jb + nbuf, pb)

            return carry

        lax.fori_loop(0, cps // nbuf, group_body, 0)
        s_wait(nbuf - 1)
        plsc.subcore_barrier()

        pltpu.sync_copy(agg_sh.at[pl.ds(s * rpt, rpt)],
                        out_hbm.at[c, pl.ds(s * rpt, rpt)])

    return body(x2, col3, row3, ev3, zeros)


def _linear_body(p_ref, w_ref, b_ref, o_ref):
    s = jnp.concatenate([p_ref[0], p_ref[1]], axis=1)
    acc = lax.dot_general(s, w_ref[...], (((1,), (1,)), ((), ())),
                          preferred_element_type=jnp.float32)
    o_ref[...] = acc + b_ref[...]


def _tc_linear(n, d, partials, w, b):
    rb = 1000
    dh = d // _NC
    return pl.pallas_call(
        _linear_body,
        grid=(n // rb,),
        in_specs=[
            pl.BlockSpec((_NC, rb, dh), lambda i: (0, i, 0)),
            pl.BlockSpec((d, d), lambda i: (0, 0)),
            pl.BlockSpec((1, d), lambda i: (0, 0)),
        ],
        out_specs=pl.BlockSpec((rb, d), lambda i: (i, 0)),
        out_shape=jax.ShapeDtypeStruct((n, d), jnp.float32),
    )(partials, w, b.reshape(1, d))


def kernel(x, edge_index, edge_values, W, b):
    n, d = x.shape
    e = edge_values.shape[0]
    dh = d // _NC
    cps = e // (_NS * _CW)
    npad = ((n + _NS * 8 - 1) // (_NS * 8)) * (_NS * 8)
    row3 = edge_index[0].reshape(_NS, cps, _CW)
    col3 = edge_index[1].reshape(_NS, cps, _CW)
    ev3 = edge_values.reshape(_NS, cps, _CW)
    x2 = jnp.stack([x[:, :dh], x[:, dh:]])
    zeros = jnp.zeros((npad, dh), jnp.float32)
    partials = _sc_aggregate(npad, d, x2, col3, row3, ev3, zeros)
    return _tc_linear(n, d, partials, W, b)
